# baseline (speedup 1.0000x reference)
import functools

import jax
import jax.numpy as jnp
from jax import lax
from jax.experimental import pallas as pl
from jax.experimental.pallas import tpu as pltpu
from jax.experimental.pallas import tpu_sc as plsc

F = 64
L = 16


@jax.jit
def kernel(user, item, user_factors, item_factors):
    B = user.shape[0]
    info = plsc.get_sparse_core_info()
    NW = info.num_cores * info.num_subcores
    NC = info.num_cores
    b_per_w = B // NW
    n_groups = b_per_w // L

    mesh = plsc.VectorSubcoreMesh(core_axis_name="c", subcore_axis_name="s")

    @functools.partial(
        pl.kernel,
        mesh=mesh,
        out_type=jax.ShapeDtypeStruct((B,), jnp.float32),
        compiler_params=pltpu.CompilerParams(
            needs_layout_passes=False, use_tc_tiling_on_sc=False
        ),
        scratch_types=[
            pltpu.VMEM((b_per_w,), jnp.int32),
            pltpu.VMEM((b_per_w,), jnp.int32),
            pltpu.VMEM((b_per_w, F), jnp.float32),
            pltpu.VMEM((b_per_w, F), jnp.float32),
            pltpu.VMEM((b_per_w,), jnp.float32),
            pltpu.SemaphoreType.DMA,
            pltpu.SemaphoreType.DMA,
        ],
    )
    def sc_kernel(user_hbm, item_hbm, uf_hbm, if_hbm, out_hbm,
                  uidx_v, iidx_v, urows_v, irows_v, out_v, sem_u, sem_i):
        wid = lax.axis_index("s") * NC + lax.axis_index("c")
        base = wid * b_per_w
        pltpu.sync_copy(user_hbm.at[pl.ds(base, b_per_w)], uidx_v)
        pltpu.sync_copy(item_hbm.at[pl.ds(base, b_per_w)], iidx_v)
        cp_u = pltpu.async_copy(uf_hbm.at[uidx_v], urows_v, sem_u)
        cp_i = pltpu.async_copy(if_hbm.at[iidx_v], irows_v, sem_i)
        cp_u.wait()
        cp_i.wait()

        def body(g, carry):
            rows = g * L + lax.iota(jnp.int32, L)
            accs = [jnp.zeros((L,), jnp.float32) for _ in range(4)]
            for f in range(F):
                cols = jnp.full((L,), f, jnp.int32)
                uc = plsc.load_gather(urows_v, [rows, cols])
                vc = plsc.load_gather(irows_v, [rows, cols])
                accs[f % 4] = accs[f % 4] + uc * vc
            acc = (accs[0] + accs[1]) + (accs[2] + accs[3])
            out_v[pl.ds(g * L, L)] = acc
            return carry

        lax.fori_loop(0, n_groups, body, 0)
        pltpu.sync_copy(out_v, out_hbm.at[pl.ds(base, b_per_w)])

    return sc_kernel(user, item, user_factors, item_factors)


# speedup vs baseline: 1.5303x; 1.5303x over previous
import functools

import jax
import jax.numpy as jnp
from jax import lax
from jax.experimental import pallas as pl
from jax.experimental.pallas import tpu as pltpu
from jax.experimental.pallas import tpu_sc as plsc

F = 64
L = 16
CHUNK = 256


@jax.jit
def kernel(user, item, user_factors, item_factors):
    B = user.shape[0]
    info = plsc.get_sparse_core_info()
    NC = info.num_cores
    NW = NC * info.num_subcores
    b_per_w = B // NW
    n_chunks = b_per_w // CHUNK
    groups_per_chunk = CHUNK // L

    mesh = plsc.VectorSubcoreMesh(core_axis_name="c", subcore_axis_name="s")

    @functools.partial(
        pl.kernel,
        mesh=mesh,
        out_type=jax.ShapeDtypeStruct((B,), jnp.float32),
        compiler_params=pltpu.CompilerParams(needs_layout_passes=False),
        scratch_types=[
            pltpu.VMEM((b_per_w,), jnp.int32),
            pltpu.VMEM((b_per_w,), jnp.int32),
            pltpu.SMEM((b_per_w,), jnp.int32),
            pltpu.SMEM((b_per_w,), jnp.int32),
            pltpu.VMEM((CHUNK, F), jnp.float32),
            pltpu.VMEM((CHUNK, F), jnp.float32),
            pltpu.VMEM((b_per_w,), jnp.float32),
            pltpu.SemaphoreType.DMA,
            pltpu.SemaphoreType.DMA,
        ],
    )
    def sc_kernel(user_hbm, item_hbm, uf_hbm, if_hbm, out_hbm,
                  uidx_v, iidx_v, uidx_s, iidx_s, urows, irows, out_v,
                  sem_u, sem_i):
        wid = lax.axis_index("s") * NC + lax.axis_index("c")
        base = wid * b_per_w
        pltpu.sync_copy(user_hbm.at[pl.ds(base, b_per_w)], uidx_v)
        pltpu.sync_copy(item_hbm.at[pl.ds(base, b_per_w)], iidx_v)

        lanes = lax.iota(jnp.int32, L)

        def stage(g, carry):
            uvec = uidx_v[pl.ds(g * L, L)]
            ivec = iidx_v[pl.ds(g * L, L)]
            zero = jnp.zeros((L,), jnp.int32)
            for j in range(L):
                sel = lanes == j
                uidx_s[g * L + j] = jnp.sum(jnp.where(sel, uvec, zero))
                iidx_s[g * L + j] = jnp.sum(jnp.where(sel, ivec, zero))
            return carry

        lax.fori_loop(0, b_per_w // L, stage, 0)

        for c in range(n_chunks):

            def issue(i, carry):
                ur = uidx_s[c * CHUNK + i]
                ir = iidx_s[c * CHUNK + i]
                pltpu.async_copy(uf_hbm.at[pl.ds(ur, 1)],
                                 urows.at[pl.ds(i, 1)], sem_u)
                pltpu.async_copy(if_hbm.at[pl.ds(ir, 1)],
                                 irows.at[pl.ds(i, 1)], sem_i)
                return carry

            lax.fori_loop(0, CHUNK, issue, 0)
            pltpu.make_async_copy(uf_hbm.at[pl.ds(0, CHUNK)], urows,
                                  sem_u).wait()
            pltpu.make_async_copy(if_hbm.at[pl.ds(0, CHUNK)], irows,
                                  sem_i).wait()

            for g in range(groups_per_chunk):
                rows = g * L + lax.iota(jnp.int32, L)
                accs = [jnp.zeros((L,), jnp.float32) for _ in range(4)]
                for f in range(F):
                    colf = jnp.full((L,), f, jnp.int32)
                    uc = plsc.load_gather(urows, [rows, colf])
                    vc = plsc.load_gather(irows, [rows, colf])
                    accs[f % 4] = accs[f % 4] + uc * vc
                acc = (accs[0] + accs[1]) + (accs[2] + accs[3])
                out_v[pl.ds(c * CHUNK + g * L, L)] = acc

        pltpu.sync_copy(out_v, out_hbm.at[pl.ds(base, b_per_w)])

    return sc_kernel(user, item, user_factors, item_factors)


# speedup vs baseline: 2.1492x; 1.4044x over previous
import functools

import jax
import jax.numpy as jnp
from jax import lax
from jax.experimental import pallas as pl
from jax.experimental.pallas import tpu as pltpu
from jax.experimental.pallas import tpu_sc as plsc

F = 64
L = 16
TC = 128
CHUNK = 256


def _stage_scalars(idx_v, dst_s, n, transform):
    lanes = lax.iota(jnp.int32, L)

    def stage(g, carry):
        vec = transform(idx_v[pl.ds(g * L, L)])
        zero = jnp.zeros((L,), jnp.int32)
        for j in range(L):
            dst_s[g * L + j] = jnp.sum(jnp.where(lanes == j, vec, zero))
        return carry

    lax.fori_loop(0, n // L, stage, 0)


@jax.jit
def kernel(user, item, user_factors, item_factors):
    B = user.shape[0]
    info = plsc.get_sparse_core_info()
    NC = info.num_cores
    NW = NC * info.num_subcores
    b_per_w = B // NW

    if_t = item_factors.T
    mesh = plsc.VectorSubcoreMesh(core_axis_name="c", subcore_axis_name="s")

    @functools.partial(
        pl.kernel,
        mesh=mesh,
        out_type=jax.ShapeDtypeStruct((B, F), jnp.float32),
        compiler_params=pltpu.CompilerParams(needs_layout_passes=False),
        scratch_types=[
            pltpu.VMEM((b_per_w,), jnp.int32),
            pltpu.SMEM((b_per_w,), jnp.int32),
            pltpu.SMEM((b_per_w,), jnp.int32),
            pltpu.VMEM((F, TC), jnp.float32),
            pltpu.VMEM((F, TC), jnp.float32),
            pltpu.VMEM((b_per_w, F), jnp.float32),
            pltpu.SemaphoreType.DMA,
            pltpu.SemaphoreType.DMA,
        ],
    )
    def gather_cols(item_hbm, ift_hbm, rows_hbm,
                    iidx_v, blk_s, sub_s, buf_a, buf_b, rows_v, sem_a, sem_b):
        wid = lax.axis_index("s") * NC + lax.axis_index("c")
        base = wid * b_per_w
        pltpu.sync_copy(item_hbm.at[pl.ds(base, b_per_w)], iidx_v)
        _stage_scalars(iidx_v, blk_s, b_per_w,
                       lambda v: lax.shift_right_logical(v, 7))
        _stage_scalars(iidx_v, sub_s, b_per_w,
                       lambda v: jnp.bitwise_and(v, jnp.full((L,), TC - 1,
                                                             jnp.int32)))

        def fetch(e, buf, sem):
            off = pl.multiple_of(blk_s[e] * TC, TC)
            return pltpu.async_copy(ift_hbm.at[:, pl.ds(off, TC)], buf, sem)

        def extract(e, buf):
            sub = jnp.full((L,), sub_s[e], jnp.int32)
            erow = jnp.full((L,), e, jnp.int32)
            for k in range(F // L):
                fvec = k * L + lax.iota(jnp.int32, L)
                vals = plsc.load_gather(buf, [fvec, sub])
                plsc.store_scatter(rows_v, [erow, fvec], vals)

        fetch(0, buf_a, sem_a)

        def body(p, carry):
            e = p * 2
            fetch(e + 1, buf_b, sem_b)
            pltpu.make_async_copy(ift_hbm.at[:, pl.ds(0, TC)], buf_a,
                                  sem_a).wait()
            extract(e, buf_a)

            @pl.when(p + 1 < b_per_w // 2)
            def _():
                fetch(e + 2, buf_a, sem_a)

            pltpu.make_async_copy(ift_hbm.at[:, pl.ds(0, TC)], buf_b,
                                  sem_b).wait()
            extract(e + 1, buf_b)
            return carry

        lax.fori_loop(0, b_per_w // 2, body, 0)
        pltpu.sync_copy(rows_v, rows_hbm.at[pl.ds(base, b_per_w), :])

    @functools.partial(
        pl.kernel,
        mesh=mesh,
        out_type=jax.ShapeDtypeStruct((B,), jnp.float32),
        compiler_params=pltpu.CompilerParams(needs_layout_passes=False),
        scratch_types=[
            pltpu.VMEM((b_per_w,), jnp.int32),
            pltpu.SMEM((b_per_w,), jnp.int32),
            pltpu.VMEM((CHUNK, F), jnp.float32),
            pltpu.VMEM((b_per_w, F), jnp.float32),
            pltpu.VMEM((b_per_w,), jnp.float32),
            pltpu.SemaphoreType.DMA,
            pltpu.SemaphoreType.DMA,
        ],
    )
    def dot_rows(user_hbm, uf_hbm, vrows_hbm, out_hbm,
                 uidx_v, uidx_s, urows, vrows_v, out_v, sem_u, sem_v):
        wid = lax.axis_index("s") * NC + lax.axis_index("c")
        base = wid * b_per_w
        cp_v = pltpu.async_copy(vrows_hbm.at[pl.ds(base, b_per_w), :],
                                vrows_v, sem_v)
        pltpu.sync_copy(user_hbm.at[pl.ds(base, b_per_w)], uidx_v)
        _stage_scalars(uidx_v, uidx_s, b_per_w, lambda v: v)
        cp_v.wait()

        for c in range(b_per_w // CHUNK):

            def issue(i, carry):
                ur = uidx_s[c * CHUNK + i]
                pltpu.async_copy(uf_hbm.at[pl.ds(ur, 1)],
                                 urows.at[pl.ds(i, 1)], sem_u)
                return carry

            lax.fori_loop(0, CHUNK, issue, 0)
            pltpu.make_async_copy(uf_hbm.at[pl.ds(0, CHUNK)], urows,
                                  sem_u).wait()

            for g in range(CHUNK // L):
                rows = g * L + lax.iota(jnp.int32, L)
                vrow = c * CHUNK + g * L + lax.iota(jnp.int32, L)
                accs = [jnp.zeros((L,), jnp.float32) for _ in range(4)]
                for f in range(F):
                    colf = jnp.full((L,), f, jnp.int32)
                    uc = plsc.load_gather(urows, [rows, colf])
                    vc = plsc.load_gather(vrows_v, [vrow, colf])
                    accs[f % 4] = accs[f % 4] + uc * vc
                acc = (accs[0] + accs[1]) + (accs[2] + accs[3])
                out_v[pl.ds(c * CHUNK + g * L, L)] = acc

        pltpu.sync_copy(out_v, out_hbm.at[pl.ds(base, b_per_w)])

    item_rows = gather_cols(item, if_t)
    return dot_rows(user, user_factors, item_rows)


# speedup vs baseline: 2.3322x; 1.0852x over previous
import functools

import jax
import jax.numpy as jnp
from jax import lax
from jax.experimental import pallas as pl
from jax.experimental.pallas import tpu as pltpu
from jax.experimental.pallas import tpu_sc as plsc

F = 64
L = 16
TC = 128
CHUNK = 256


def _stage_scalars(idx_v, dst_s, n, transform):
    lanes = lax.iota(jnp.int32, L)

    def stage(g, carry):
        vec = transform(idx_v[pl.ds(g * L, L)])
        zero = jnp.zeros((L,), jnp.int32)
        for j in range(L):
            dst_s[g * L + j] = jnp.sum(jnp.where(lanes == j, vec, zero))
        return carry

    lax.fori_loop(0, n // L, stage, 0)


@jax.jit
def kernel(user, item, user_factors, item_factors):
    B = user.shape[0]
    info = plsc.get_sparse_core_info()
    NC = info.num_cores
    NW = NC * info.num_subcores
    b_per_w = B // NW

    if_t = item_factors.T
    mesh = plsc.VectorSubcoreMesh(core_axis_name="c", subcore_axis_name="s")

    G = 4

    @functools.partial(
        pl.kernel,
        mesh=mesh,
        out_type=jax.ShapeDtypeStruct((B // 2, 2 * F), jnp.float32),
        compiler_params=pltpu.CompilerParams(needs_layout_passes=False),
        scratch_types=[
            pltpu.VMEM((b_per_w,), jnp.int32),
            pltpu.SMEM((b_per_w,), jnp.int32),
            pltpu.SMEM((b_per_w,), jnp.int32),
            pltpu.VMEM((G, F, TC), jnp.float32),
            pltpu.VMEM((G, F, TC), jnp.float32),
            pltpu.VMEM((b_per_w // 2, 2 * F), jnp.float32),
            pltpu.SemaphoreType.DMA,
            pltpu.SemaphoreType.DMA,
        ],
    )
    def gather_cols(item_hbm, ift_hbm, rows_hbm,
                    iidx_v, blk_s, sub_s, buf_a, buf_b, rows_v, sem_a, sem_b):
        wid = lax.axis_index("s") * NC + lax.axis_index("c")
        base = wid * b_per_w
        pltpu.sync_copy(item_hbm.at[pl.ds(base, b_per_w)], iidx_v)
        _stage_scalars(iidx_v, blk_s, b_per_w,
                       lambda v: lax.shift_right_logical(v, 7))
        _stage_scalars(iidx_v, sub_s, b_per_w,
                       lambda v: jnp.bitwise_and(v, jnp.full((L,), TC - 1,
                                                             jnp.int32)))

        def fetch_group(e0, buf, sem):
            for j in range(G):
                off = pl.multiple_of(blk_s[e0 + j] * TC, TC)
                pltpu.async_copy(ift_hbm.at[:, pl.ds(off, TC)],
                                 buf.at[j], sem)

        def drain_group(buf, sem):
            for j in range(G):
                pltpu.make_async_copy(ift_hbm.at[:, pl.ds(0, TC)],
                                      buf.at[j], sem).wait()

        def extract_group(e0, buf):
            for j in range(G):
                e = e0 + j
                sub = jnp.full((L,), sub_s[e], jnp.int32)
                jvec = jnp.full((L,), j, jnp.int32)
                erow = jnp.full((L,), lax.shift_right_logical(e, 1),
                                jnp.int32)
                cbase = jnp.bitwise_and(e, 1) * F
                for k in range(F // L):
                    fvec = k * L + lax.iota(jnp.int32, L)
                    vals = plsc.load_gather(buf, [jvec, fvec, sub])
                    plsc.store_scatter(rows_v, [erow, cbase + fvec], vals)

        n_pairs = b_per_w // (2 * G)
        fetch_group(0, buf_a, sem_a)

        def body(p, carry):
            e = p * 2 * G
            fetch_group(e + G, buf_b, sem_b)
            drain_group(buf_a, sem_a)
            extract_group(e, buf_a)

            @pl.when(p + 1 < n_pairs)
            def _():
                fetch_group(e + 2 * G, buf_a, sem_a)

            drain_group(buf_b, sem_b)
            extract_group(e + G, buf_b)
            return carry

        lax.fori_loop(0, n_pairs, body, 0)
        pltpu.sync_copy(rows_v,
                        rows_hbm.at[pl.ds(pl.multiple_of(base // 2, 8), b_per_w // 2), :])

    @functools.partial(
        pl.kernel,
        mesh=mesh,
        out_type=jax.ShapeDtypeStruct((B,), jnp.float32),
        compiler_params=pltpu.CompilerParams(needs_layout_passes=False),
        scratch_types=[
            pltpu.VMEM((b_per_w,), jnp.int32),
            pltpu.SMEM((b_per_w,), jnp.int32),
            pltpu.VMEM((CHUNK, F), jnp.float32),
            pltpu.VMEM((b_per_w // 2, 2 * F), jnp.float32),
            pltpu.VMEM((b_per_w,), jnp.float32),
            pltpu.SemaphoreType.DMA,
            pltpu.SemaphoreType.DMA,
        ],
    )
    def dot_rows(user_hbm, uf_hbm, vrows_hbm, out_hbm,
                 uidx_v, uidx_s, urows, vrows_v, out_v, sem_u, sem_v):
        wid = lax.axis_index("s") * NC + lax.axis_index("c")
        base = wid * b_per_w
        cp_v = pltpu.async_copy(
            vrows_hbm.at[pl.ds(pl.multiple_of(base // 2, 8), b_per_w // 2), :], vrows_v, sem_v)
        pltpu.sync_copy(user_hbm.at[pl.ds(base, b_per_w)], uidx_v)
        _stage_scalars(uidx_v, uidx_s, b_per_w, lambda v: v)
        cp_v.wait()

        for c in range(b_per_w // CHUNK):

            def issue(i, carry):
                ur = uidx_s[c * CHUNK + i]
                pltpu.async_copy(uf_hbm.at[pl.ds(ur, 1)],
                                 urows.at[pl.ds(i, 1)], sem_u)
                return carry

            lax.fori_loop(0, CHUNK, issue, 0)
            pltpu.make_async_copy(uf_hbm.at[pl.ds(0, CHUNK)], urows,
                                  sem_u).wait()

            for g in range(CHUNK // L):
                rows = g * L + lax.iota(jnp.int32, L)
                il = c * CHUNK + g * L + lax.iota(jnp.int32, L)
                vrow = lax.shift_right_logical(il, 1)
                vbase = jnp.bitwise_and(il, jnp.full((L,), 1, jnp.int32)) * F
                accs = [jnp.zeros((L,), jnp.float32) for _ in range(4)]
                for f in range(F):
                    colf = jnp.full((L,), f, jnp.int32)
                    uc = plsc.load_gather(urows, [rows, colf])
                    vc = plsc.load_gather(vrows_v, [vrow, vbase + colf])
                    accs[f % 4] = accs[f % 4] + uc * vc
                acc = (accs[0] + accs[1]) + (accs[2] + accs[3])
                out_v[pl.ds(c * CHUNK + g * L, L)] = acc

        pltpu.sync_copy(out_v, out_hbm.at[pl.ds(base, b_per_w)])

    item_rows = gather_cols(item, if_t)
    return dot_rows(user, user_factors, item_rows)
